# baseline (speedup 1.0000x reference)
import functools

import jax
import jax.numpy as jnp
from jax import lax
from jax.experimental import pallas as pl
from jax.experimental.pallas import tpu as pltpu
from jax.experimental.pallas import tpu_sc as plsc

N = 10000
E = 320000
D = 128
NPAD = 10240
NB = NPAD // 128
CH = 128
NC = 2
NS = 16
NW = NC * NS
MAXCH = 80
EPAD = NW * MAXCH * CH
PER_TILE_N = NPAD // NS
ZR = 64

_mesh = functools.partial(
    plsc.VectorSubcoreMesh, core_axis_name="c", subcore_axis_name="s")


def _deg_body(dst_hbm, ones_hbm, zeros_hbm, out_hbm, idx0, idx1, onesv,
              stagev, deg_sp, sem0, sem1):
    c = lax.axis_index("c")
    s = lax.axis_index("s")
    wid = c * NS + s
    pltpu.sync_copy(ones_hbm, onesv)
    pltpu.sync_copy(zeros_hbm, stagev)
    pltpu.sync_copy(stagev, deg_sp.at[pl.ds(s * PER_TILE_N, PER_TILE_N)])
    plsc.subcore_barrier()
    base = wid * MAXCH * CH
    idxs = (idx0, idx1)
    sems = (sem0, sem1)

    def add(k, p):
        pltpu.sync_copy(dst_hbm.at[pl.ds(base + k * CH, CH)], idxs[p])
        pltpu.async_copy(onesv, deg_sp.at[idxs[p]], sems[p], add=True)

    def drain(p):
        pltpu.make_async_copy(onesv, deg_sp.at[idxs[p]], sems[p]).wait()

    add(0, 0)

    def body(gg, carry):
        k0 = 2 * gg
        add(k0 + 1, 1)
        drain(0)
        add(k0 + 2, 0)
        drain(1)
        return carry

    lax.fori_loop(0, MAXCH // 2 - 1, body, 0)
    add(MAXCH - 1, 1)
    drain(0)
    drain(1)
    plsc.subcore_barrier()
    pltpu.sync_copy(deg_sp.at[pl.ds(s * PER_TILE_N, PER_TILE_N)], stagev)
    pltpu.sync_copy(stagev, out_hbm.at[c, pl.ds(s * PER_TILE_N, PER_TILE_N)])


def _deg_call(dst1, ones1, zeros1):
    return pl.kernel(
        _deg_body,
        out_type=jax.ShapeDtypeStruct((NC, NPAD), jnp.float32),
        mesh=_mesh(),
        scratch_types=[
            pltpu.VMEM((CH,), jnp.int32),
            pltpu.VMEM((CH,), jnp.int32),
            pltpu.VMEM((CH,), jnp.float32),
            pltpu.VMEM((PER_TILE_N,), jnp.float32),
            pltpu.VMEM_SHARED((NPAD,), jnp.float32),
            pltpu.SemaphoreType.DMA,
            pltpu.SemaphoreType.DMA,
        ],
    )(dst1, ones1, zeros1)


def _dot(a, b):
    return lax.dot_general(a, b, (((1,), (0,)), ((), ())),
                           preferred_element_type=jnp.float32)


def _mm_body(x_ref, w_ref, dp_ref, eye_ref, xw_ref, y_ref):
    deg = dp_ref[0] + dp_ref[1] + 1.0
    dis = lax.rsqrt(deg)
    xw = lax.dot_general(x_ref[...], w_ref[...], (((1,), (1,)), ((), ())),
                         preferred_element_type=jnp.float32)
    xw_ref[...] = xw
    eye = eye_ref[...]
    for j in range(8):
        diagm = dis[j:j + 1, :] * eye
        y_ref[128 * j:128 * (j + 1), :] = _dot(diagm, xw[128 * j:128 * (j + 1), :])


def _mm_call(x_pad, w, dp3, eye):
    return pl.pallas_call(
        _mm_body,
        grid=(10,),
        in_specs=[
            pl.BlockSpec((1024, D), lambda i: (i, 0)),
            pl.BlockSpec((D, D), lambda i: (0, 0)),
            pl.BlockSpec((NC, 8, 128), lambda i: (0, i, 0)),
            pl.BlockSpec((D, D), lambda i: (0, 0)),
        ],
        out_specs=[
            pl.BlockSpec((1024, D), lambda i: (i, 0)),
            pl.BlockSpec((1024, D), lambda i: (i, 0)),
        ],
        out_shape=[
            jax.ShapeDtypeStruct((NPAD, D), jnp.float32),
            jax.ShapeDtypeStruct((NPAD, D), jnp.float32),
        ],
    )(x_pad, w, dp3, eye)


def _edge_body(y_hbm, src_hbm, dst_hbm, zeros_hbm, out_hbm, src0, src1, dst0,
               dst1, rows0, rows1, stage, acc_sp, semi0, semi1, semid0, semid1,
               semg0, semg1):
    c = lax.axis_index("c")
    s = lax.axis_index("s")
    wid = c * NS + s
    pltpu.sync_copy(zeros_hbm, stage)
    for k in range(PER_TILE_N // ZR):
        pltpu.sync_copy(stage, acc_sp.at[pl.ds(s * PER_TILE_N + k * ZR, ZR)])
    plsc.subcore_barrier()

    base = wid * MAXCH * CH
    srcs = (src0, src1)
    dsts = (dst0, dst1)
    rows = (rows0, rows1)
    semis = (semi0, semi1)
    semid = (semid0, semid1)
    semg = (semg0, semg1)

    def load_src(k, p):
        pltpu.async_copy(src_hbm.at[pl.ds(base + k * CH, CH)], srcs[p],
                         semis[p])

    def load_dst(k, p):
        pltpu.async_copy(dst_hbm.at[pl.ds(base + k * CH, CH)], dsts[p],
                         semid[p])

    def gather(k, p):
        pltpu.make_async_copy(src_hbm.at[pl.ds(base + k * CH, CH)], srcs[p],
                              semis[p]).wait()
        pltpu.async_copy(y_hbm.at[srcs[p]], rows[p], semg[p])

    def wait_gather(p):
        pltpu.make_async_copy(y_hbm.at[srcs[p]], rows[p], semg[p]).wait()

    def scatter(k, p):
        pltpu.make_async_copy(dst_hbm.at[pl.ds(base + k * CH, CH)], dsts[p],
                              semid[p]).wait()
        pltpu.sync_copy(rows[p], acc_sp.at[dsts[p]], add=True)

    load_src(0, 0)
    load_dst(0, 0)
    load_src(1, 1)
    load_dst(1, 1)
    gather(0, 0)

    def body(gg, carry):
        k0 = 2 * gg
        k1 = 2 * gg + 1
        gather(k1, 1)
        wait_gather(0)
        load_src(k0 + 2, 0)
        scatter(k0, 0)
        load_dst(k0 + 2, 0)
        gather(k0 + 2, 0)
        wait_gather(1)
        load_src(k1 + 2, 1)
        scatter(k1, 1)
        load_dst(k1 + 2, 1)
        return carry

    lax.fori_loop(0, MAXCH // 2 - 1, body, 0)
    gather(MAXCH - 1, 1)
    wait_gather(0)
    scatter(MAXCH - 2, 0)
    wait_gather(1)
    scatter(MAXCH - 1, 1)
    plsc.subcore_barrier()
    for k in range(PER_TILE_N // ZR):
        pltpu.sync_copy(acc_sp.at[pl.ds(s * PER_TILE_N + k * ZR, ZR)], stage)
        pltpu.sync_copy(stage, out_hbm.at[c, pl.ds(s * PER_TILE_N + k * ZR, ZR)])


def _edge_call(y, src1d, dst1d, zeros2):
    return pl.kernel(
        _edge_body,
        out_type=jax.ShapeDtypeStruct((NC, NPAD, D), jnp.float32),
        mesh=_mesh(),
        scratch_types=[
            pltpu.VMEM((CH,), jnp.int32),
            pltpu.VMEM((CH,), jnp.int32),
            pltpu.VMEM((CH,), jnp.int32),
            pltpu.VMEM((CH,), jnp.int32),
            pltpu.VMEM((CH, D), jnp.float32),
            pltpu.VMEM((CH, D), jnp.float32),
            pltpu.VMEM((ZR, D), jnp.float32),
            pltpu.VMEM_SHARED((NPAD, D), jnp.float32),
            pltpu.SemaphoreType.DMA,
            pltpu.SemaphoreType.DMA,
            pltpu.SemaphoreType.DMA,
            pltpu.SemaphoreType.DMA,
            pltpu.SemaphoreType.DMA,
            pltpu.SemaphoreType.DMA,
        ],
    )(y, src1d, dst1d, zeros2)


def _ep_body(acc_ref, xw_ref, dp_ref, b_ref, eye_ref, out_ref):
    deg = dp_ref[0] + dp_ref[1] + 1.0
    dis = lax.rsqrt(deg)
    invd = 1.0 / deg
    eye = eye_ref[...]
    acc = acc_ref[0] + acc_ref[1]
    for j in range(8):
        sl = slice(128 * j, 128 * (j + 1))
        dism = dis[j:j + 1, :] * eye
        invm = invd[j:j + 1, :] * eye
        h = _dot(dism, acc[sl, :]) + _dot(invm, xw_ref[sl, :]) + b_ref[0:1, :]
        out_ref[sl, :] = jnp.maximum(h, 0.0)


def _ep_call(accs, xw, dp3, b8, eye):
    return pl.pallas_call(
        _ep_body,
        grid=(10,),
        in_specs=[
            pl.BlockSpec((NC, 1024, D), lambda i: (0, i, 0)),
            pl.BlockSpec((1024, D), lambda i: (i, 0)),
            pl.BlockSpec((NC, 8, 128), lambda i: (0, i, 0)),
            pl.BlockSpec((8, D), lambda i: (0, 0)),
            pl.BlockSpec((D, D), lambda i: (0, 0)),
        ],
        out_specs=pl.BlockSpec((1024, D), lambda i: (i, 0)),
        out_shape=jax.ShapeDtypeStruct((NPAD, D), jnp.float32),
    )(accs, xw, dp3, b8, eye)


def kernel(x, edge_index, W, b):
    pad = jnp.full((EPAD - E,), NPAD - 1, jnp.int32)
    src1d = jnp.concatenate([edge_index[0], pad])
    dst1d = jnp.concatenate([edge_index[1], pad])
    x_pad = jnp.pad(x, ((0, NPAD - N), (0, 0)))
    ones1 = jnp.ones((CH,), jnp.float32)
    zeros1 = jnp.zeros((PER_TILE_N,), jnp.float32)
    zeros2 = jnp.zeros((ZR, D), jnp.float32)
    b8 = jnp.broadcast_to(b[None, :], (8, D))
    eye = jnp.eye(D, dtype=jnp.float32)

    dpart = _deg_call(dst1d, ones1, zeros1)
    dp3 = dpart.reshape(NC, NB, 128)
    xw, y = _mm_call(x_pad, W, dp3, eye)
    accs = _edge_call(y, src1d, dst1d, zeros2)
    out = _ep_call(accs, xw, dp3, b8, eye)
    return out[:N]


# speedup vs baseline: 2.7667x; 2.7667x over previous
import functools

import jax
import jax.numpy as jnp
from jax import lax
from jax.experimental import pallas as pl
from jax.experimental.pallas import tpu as pltpu
from jax.experimental.pallas import tpu_sc as plsc

N = 10000
E = 320000
D = 128
NPAD = 10240
NB = NPAD // 128
CH = 128
NC = 2
NS = 16
NW = NC * NS
MAXCH = 80
EPAD = NW * MAXCH * CH
PER_TILE_N = NPAD // NS
ZR = 64

_mesh = functools.partial(
    plsc.VectorSubcoreMesh, core_axis_name="c", subcore_axis_name="s")


def _deg_body(dst_hbm, ones_hbm, zeros_hbm, out_hbm, idx0, idx1, onesv,
              stagev, deg_sp, sem0, sem1):
    c = lax.axis_index("c")
    s = lax.axis_index("s")
    wid = c * NS + s
    pltpu.sync_copy(ones_hbm, onesv)
    pltpu.sync_copy(zeros_hbm, stagev)
    pltpu.sync_copy(stagev, deg_sp.at[pl.ds(s * PER_TILE_N, PER_TILE_N)])
    plsc.subcore_barrier()
    base = wid * MAXCH * CH
    idxs = (idx0, idx1)
    sems = (sem0, sem1)

    def add(k, p):
        pltpu.sync_copy(dst_hbm.at[pl.ds(base + k * CH, CH)], idxs[p])
        pltpu.async_copy(onesv, deg_sp.at[idxs[p]], sems[p], add=True)

    def drain(p):
        pltpu.make_async_copy(onesv, deg_sp.at[idxs[p]], sems[p]).wait()

    add(0, 0)

    def body(gg, carry):
        k0 = 2 * gg
        add(k0 + 1, 1)
        drain(0)
        add(k0 + 2, 0)
        drain(1)
        return carry

    lax.fori_loop(0, MAXCH // 2 - 1, body, 0)
    add(MAXCH - 1, 1)
    drain(0)
    drain(1)
    plsc.subcore_barrier()
    pltpu.sync_copy(deg_sp.at[pl.ds(s * PER_TILE_N, PER_TILE_N)], stagev)
    pltpu.sync_copy(stagev, out_hbm.at[c, pl.ds(s * PER_TILE_N, PER_TILE_N)])


def _deg_call(dst1, ones1, zeros1):
    return pl.kernel(
        _deg_body,
        out_type=jax.ShapeDtypeStruct((NC, NPAD), jnp.float32),
        mesh=_mesh(),
        scratch_types=[
            pltpu.VMEM((CH,), jnp.int32),
            pltpu.VMEM((CH,), jnp.int32),
            pltpu.VMEM((CH,), jnp.float32),
            pltpu.VMEM((PER_TILE_N,), jnp.float32),
            pltpu.VMEM_SHARED((NPAD,), jnp.float32),
            pltpu.SemaphoreType.DMA,
            pltpu.SemaphoreType.DMA,
        ],
    )(dst1, ones1, zeros1)


def _dot(a, b):
    return lax.dot_general(a, b, (((1,), (0,)), ((), ())),
                           preferred_element_type=jnp.float32)


def _mm_body(x_ref, w_ref, dp_ref, eye_ref, xw_ref, y_ref):
    deg = dp_ref[0] + dp_ref[1] + 1.0
    dis = lax.rsqrt(deg)
    xw = lax.dot_general(x_ref[...], w_ref[...], (((1,), (1,)), ((), ())),
                         preferred_element_type=jnp.float32)
    xw_ref[...] = xw
    eye = eye_ref[...]
    for j in range(8):
        diagm = dis[j:j + 1, :] * eye
        y_ref[128 * j:128 * (j + 1), :] = _dot(diagm, xw[128 * j:128 * (j + 1), :])


def _mm_call(x_pad, w, dp3, eye):
    return pl.pallas_call(
        _mm_body,
        grid=(10,),
        in_specs=[
            pl.BlockSpec((1024, D), lambda i: (i, 0)),
            pl.BlockSpec((D, D), lambda i: (0, 0)),
            pl.BlockSpec((NC, 8, 128), lambda i: (0, i, 0)),
            pl.BlockSpec((D, D), lambda i: (0, 0)),
        ],
        out_specs=[
            pl.BlockSpec((1024, D), lambda i: (i, 0)),
            pl.BlockSpec((1024, D), lambda i: (i, 0)),
        ],
        out_shape=[
            jax.ShapeDtypeStruct((NPAD, D), jnp.float32),
            jax.ShapeDtypeStruct((NPAD, D), jnp.float32),
        ],
    )(x_pad, w, dp3, eye)


def _edge_body(y_hbm, src_hbm, dst_hbm, zeros_hbm, out_hbm, src0, src1, dst0,
               dst1, rows0, rows1, stage, acc_sp, semi0, semi1, semid0, semid1,
               semg0, semg1):
    c = lax.axis_index("c")
    s = lax.axis_index("s")
    wid = c * NS + s
    pltpu.sync_copy(zeros_hbm, stage)
    for k in range(PER_TILE_N // ZR):
        pltpu.sync_copy(stage, acc_sp.at[pl.ds(s * PER_TILE_N + k * ZR, ZR)])
    plsc.subcore_barrier()

    base = wid * MAXCH * CH
    srcs = (src0, src1)
    dsts = (dst0, dst1)
    rows = (rows0, rows1)
    semis = (semi0, semi1)
    semid = (semid0, semid1)
    semg = (semg0, semg1)

    def load_src(k, p):
        pltpu.async_copy(src_hbm.at[pl.ds(base + k * CH, CH)], srcs[p],
                         semis[p])

    def load_dst(k, p):
        pltpu.async_copy(dst_hbm.at[pl.ds(base + k * CH, CH)], dsts[p],
                         semid[p])

    def gather(k, p):
        pltpu.make_async_copy(src_hbm.at[pl.ds(base + k * CH, CH)], srcs[p],
                              semis[p]).wait()
        pltpu.async_copy(y_hbm.at[srcs[p]], rows[p], semg[p])

    def wait_gather(p):
        pltpu.make_async_copy(y_hbm.at[srcs[p]], rows[p], semg[p]).wait()

    def scatter(k, p):
        pltpu.make_async_copy(dst_hbm.at[pl.ds(base + k * CH, CH)], dsts[p],
                              semid[p]).wait()
        pltpu.sync_copy(rows[p], acc_sp.at[dsts[p]], add=True)

    load_src(0, 0)
    load_dst(0, 0)
    load_src(1, 1)
    load_dst(1, 1)
    gather(0, 0)

    def body(gg, carry):
        k0 = 2 * gg
        k1 = 2 * gg + 1
        gather(k1, 1)
        wait_gather(0)
        load_src(k0 + 2, 0)
        scatter(k0, 0)
        load_dst(k0 + 2, 0)
        gather(k0 + 2, 0)
        wait_gather(1)
        load_src(k1 + 2, 1)
        scatter(k1, 1)
        load_dst(k1 + 2, 1)
        return carry

    lax.fori_loop(0, MAXCH // 2 - 1, body, 0)
    gather(MAXCH - 1, 1)
    wait_gather(0)
    scatter(MAXCH - 2, 0)
    wait_gather(1)
    scatter(MAXCH - 1, 1)
    plsc.subcore_barrier()
    for k in range(PER_TILE_N // ZR):
        pltpu.sync_copy(acc_sp.at[pl.ds(s * PER_TILE_N + k * ZR, ZR)], stage)
        pltpu.sync_copy(stage, out_hbm.at[c, pl.ds(s * PER_TILE_N + k * ZR, ZR)])


def _edge_call(y, src1d, dst1d, zeros2):
    return pl.kernel(
        _edge_body,
        out_type=jax.ShapeDtypeStruct((NC, NPAD, D), jnp.float32),
        mesh=_mesh(),
        scratch_types=[
            pltpu.VMEM((CH,), jnp.int32),
            pltpu.VMEM((CH,), jnp.int32),
            pltpu.VMEM((CH,), jnp.int32),
            pltpu.VMEM((CH,), jnp.int32),
            pltpu.VMEM((CH, D), jnp.float32),
            pltpu.VMEM((CH, D), jnp.float32),
            pltpu.VMEM((ZR, D), jnp.float32),
            pltpu.VMEM_SHARED((NPAD, D), jnp.float32),
            pltpu.SemaphoreType.DMA,
            pltpu.SemaphoreType.DMA,
            pltpu.SemaphoreType.DMA,
            pltpu.SemaphoreType.DMA,
            pltpu.SemaphoreType.DMA,
            pltpu.SemaphoreType.DMA,
        ],
    )(y, src1d, dst1d, zeros2)


def _ep_body(acc_ref, xw_ref, dp_ref, b_ref, eye_ref, out_ref):
    deg = dp_ref[0] + dp_ref[1] + 1.0
    dis = lax.rsqrt(deg)
    invd = 1.0 / deg
    eye = eye_ref[...]
    acc = acc_ref[0] + acc_ref[1]
    for j in range(8):
        sl = slice(128 * j, 128 * (j + 1))
        dism = dis[j:j + 1, :] * eye
        invm = invd[j:j + 1, :] * eye
        h = _dot(dism, acc[sl, :]) + _dot(invm, xw_ref[sl, :]) + b_ref[0:1, :]
        out_ref[sl, :] = jnp.maximum(h, 0.0)


def _ep_call(accs, xw, dp3, b8, eye):
    return pl.pallas_call(
        _ep_body,
        grid=(10,),
        in_specs=[
            pl.BlockSpec((NC, 1024, D), lambda i: (0, i, 0)),
            pl.BlockSpec((1024, D), lambda i: (i, 0)),
            pl.BlockSpec((NC, 8, 128), lambda i: (0, i, 0)),
            pl.BlockSpec((8, D), lambda i: (0, 0)),
            pl.BlockSpec((D, D), lambda i: (0, 0)),
        ],
        out_specs=pl.BlockSpec((1024, D), lambda i: (i, 0)),
        out_shape=jax.ShapeDtypeStruct((NPAD, D), jnp.float32),
    )(accs, xw, dp3, b8, eye)


def kernel(x, edge_index, W, b):
    pad = N + jax.lax.rem(jnp.arange(EPAD - E, dtype=jnp.int32),
                          jnp.int32(NPAD - N))
    src1d = jnp.concatenate([edge_index[0], pad])
    dst1d = jnp.concatenate([edge_index[1], pad])
    x_pad = jnp.pad(x, ((0, NPAD - N), (0, 0)))
    ones1 = jnp.ones((CH,), jnp.float32)
    zeros1 = jnp.zeros((PER_TILE_N,), jnp.float32)
    zeros2 = jnp.zeros((ZR, D), jnp.float32)
    b8 = jnp.broadcast_to(b[None, :], (8, D))
    eye = jnp.eye(D, dtype=jnp.float32)

    dpart = _deg_call(dst1d, ones1, zeros1)
    dp3 = dpart.reshape(NC, NB, 128)
    xw, y = _mm_call(x_pad, W, dp3, eye)
    accs = _edge_call(y, src1d, dst1d, zeros2)
    out = _ep_call(accs, xw, dp3, b8, eye)
    return out[:N]


# speedup vs baseline: 3.1548x; 1.1403x over previous
import functools

import jax
import jax.numpy as jnp
from jax import lax
from jax.experimental import pallas as pl
from jax.experimental.pallas import tpu as pltpu
from jax.experimental.pallas import tpu_sc as plsc

N = 10000
E = 320000
D = 128
NPAD = 10240
NB = NPAD // 128
CH = 128
NC = 2
NS = 16
NW = NC * NS
MAXCH = 80
EPAD = NW * MAXCH * CH
PER_TILE_N = NPAD // NS
ZR = 64

_mesh = functools.partial(
    plsc.VectorSubcoreMesh, core_axis_name="c", subcore_axis_name="s")


def _deg_body(dst_hbm, ones_hbm, zeros_hbm, out_hbm, idx0, idx1, idx2, idx3,
              onesv, stagev, deg_sp, semi0, semi1, semi2, semi3, sema0, sema1,
              sema2, sema3):
    c = lax.axis_index("c")
    s = lax.axis_index("s")
    wid = c * NS + s
    pltpu.sync_copy(ones_hbm, onesv)
    pltpu.sync_copy(zeros_hbm, stagev)
    pltpu.sync_copy(stagev, deg_sp.at[pl.ds(s * PER_TILE_N, PER_TILE_N)])
    plsc.subcore_barrier()
    base = wid * MAXCH * CH
    idxs = (idx0, idx1, idx2, idx3)
    semi = (semi0, semi1, semi2, semi3)
    sema = (sema0, sema1, sema2, sema3)

    def load(k, p):
        pltpu.async_copy(dst_hbm.at[pl.ds(base + k * CH, CH)], idxs[p],
                         semi[p])

    def add(k, p):
        pltpu.make_async_copy(dst_hbm.at[pl.ds(base + k * CH, CH)], idxs[p],
                              semi[p]).wait()
        pltpu.async_copy(onesv, deg_sp.at[idxs[p]], sema[p], add=True)

    def drain(p):
        pltpu.make_async_copy(onesv, deg_sp.at[idxs[p]], sema[p]).wait()

    for p in range(4):
        load(p, p)

    def body(gg, carry):
        k = 4 * gg
        for p in range(4):
            add(k + p, p)
        for p in range(4):
            drain(p)
            load(k + 4 + p, p)
        return carry

    lax.fori_loop(0, MAXCH // 4 - 1, body, 0)
    for p in range(4):
        add(MAXCH - 4 + p, p)
    for p in range(4):
        drain(p)
    plsc.subcore_barrier()
    pltpu.sync_copy(deg_sp.at[pl.ds(s * PER_TILE_N, PER_TILE_N)], stagev)
    pltpu.sync_copy(stagev, out_hbm.at[c, pl.ds(s * PER_TILE_N, PER_TILE_N)])


def _deg_call(dst1, ones1, zeros1):
    return pl.kernel(
        _deg_body,
        out_type=jax.ShapeDtypeStruct((NC, NPAD), jnp.float32),
        mesh=_mesh(),
        scratch_types=(
            [pltpu.VMEM((CH,), jnp.int32)] * 4
            + [pltpu.VMEM((CH,), jnp.float32),
               pltpu.VMEM((PER_TILE_N,), jnp.float32),
               pltpu.VMEM_SHARED((NPAD,), jnp.float32)]
            + [pltpu.SemaphoreType.DMA] * 8
        ),
    )(dst1, ones1, zeros1)


def _dot(a, b):
    return lax.dot_general(a, b, (((1,), (0,)), ((), ())),
                           preferred_element_type=jnp.float32)


def _mm_body(x_ref, w_ref, xw_ref):
    xw_ref[...] = lax.dot_general(x_ref[...], w_ref[...],
                                  (((1,), (1,)), ((), ())),
                                  preferred_element_type=jnp.float32)


def _mm_call(x_pad, w):
    return pl.pallas_call(
        _mm_body,
        grid=(10,),
        in_specs=[
            pl.BlockSpec((1024, D), lambda i: (i, 0)),
            pl.BlockSpec((D, D), lambda i: (0, 0)),
        ],
        out_specs=pl.BlockSpec((1024, D), lambda i: (i, 0)),
        out_shape=jax.ShapeDtypeStruct((NPAD, D), jnp.float32),
    )(x_pad, w)


def _scale_body(xw_ref, dp_ref, eye_ref, y_ref):
    deg = dp_ref[0] + dp_ref[1] + 1.0
    dis = lax.rsqrt(deg)
    eye = eye_ref[...]
    for j in range(8):
        diagm = dis[j:j + 1, :] * eye
        y_ref[128 * j:128 * (j + 1), :] = _dot(
            diagm, xw_ref[128 * j:128 * (j + 1), :])


def _scale_call(xw, dp3, eye):
    return pl.pallas_call(
        _scale_body,
        grid=(10,),
        in_specs=[
            pl.BlockSpec((1024, D), lambda i: (i, 0)),
            pl.BlockSpec((NC, 8, 128), lambda i: (0, i, 0)),
            pl.BlockSpec((D, D), lambda i: (0, 0)),
        ],
        out_specs=pl.BlockSpec((1024, D), lambda i: (i, 0)),
        out_shape=jax.ShapeDtypeStruct((NPAD, D), jnp.float32),
    )(xw, dp3, eye)


def _edge_body(y_hbm, src_hbm, dst_hbm, zeros_hbm, out_hbm, src0, src1, dst0,
               dst1, rows0, rows1, stage, acc_sp, semi0, semi1, semid0, semid1,
               semg0, semg1):
    c = lax.axis_index("c")
    s = lax.axis_index("s")
    wid = c * NS + s
    pltpu.sync_copy(zeros_hbm, stage)
    for k in range(PER_TILE_N // ZR):
        pltpu.sync_copy(stage, acc_sp.at[pl.ds(s * PER_TILE_N + k * ZR, ZR)])
    plsc.subcore_barrier()

    base = wid * MAXCH * CH
    srcs = (src0, src1)
    dsts = (dst0, dst1)
    rows = (rows0, rows1)
    semis = (semi0, semi1)
    semid = (semid0, semid1)
    semg = (semg0, semg1)

    def load_src(k, p):
        pltpu.async_copy(src_hbm.at[pl.ds(base + k * CH, CH)], srcs[p],
                         semis[p])

    def load_dst(k, p):
        pltpu.async_copy(dst_hbm.at[pl.ds(base + k * CH, CH)], dsts[p],
                         semid[p])

    def gather(k, p):
        pltpu.make_async_copy(src_hbm.at[pl.ds(base + k * CH, CH)], srcs[p],
                              semis[p]).wait()
        pltpu.async_copy(y_hbm.at[srcs[p]], rows[p], semg[p])

    def wait_gather(p):
        pltpu.make_async_copy(y_hbm.at[srcs[p]], rows[p], semg[p]).wait()

    def scatter(k, p):
        pltpu.make_async_copy(dst_hbm.at[pl.ds(base + k * CH, CH)], dsts[p],
                              semid[p]).wait()
        pltpu.sync_copy(rows[p], acc_sp.at[dsts[p]], add=True)

    load_src(0, 0)
    load_dst(0, 0)
    load_src(1, 1)
    load_dst(1, 1)
    gather(0, 0)

    def body(gg, carry):
        k0 = 2 * gg
        k1 = 2 * gg + 1
        gather(k1, 1)
        wait_gather(0)
        load_src(k0 + 2, 0)
        scatter(k0, 0)
        load_dst(k0 + 2, 0)
        gather(k0 + 2, 0)
        wait_gather(1)
        load_src(k1 + 2, 1)
        scatter(k1, 1)
        load_dst(k1 + 2, 1)
        return carry

    lax.fori_loop(0, MAXCH // 2 - 1, body, 0)
    gather(MAXCH - 1, 1)
    wait_gather(0)
    scatter(MAXCH - 2, 0)
    wait_gather(1)
    scatter(MAXCH - 1, 1)
    plsc.subcore_barrier()
    for k in range(PER_TILE_N // ZR):
        pltpu.sync_copy(acc_sp.at[pl.ds(s * PER_TILE_N + k * ZR, ZR)], stage)
        pltpu.sync_copy(stage, out_hbm.at[c, pl.ds(s * PER_TILE_N + k * ZR, ZR)])


def _edge_call(y, src1d, dst1d, zeros2):
    return pl.kernel(
        _edge_body,
        out_type=jax.ShapeDtypeStruct((NC, NPAD, D), jnp.float32),
        mesh=_mesh(),
        scratch_types=[
            pltpu.VMEM((CH,), jnp.int32),
            pltpu.VMEM((CH,), jnp.int32),
            pltpu.VMEM((CH,), jnp.int32),
            pltpu.VMEM((CH,), jnp.int32),
            pltpu.VMEM((CH, D), jnp.float32),
            pltpu.VMEM((CH, D), jnp.float32),
            pltpu.VMEM((ZR, D), jnp.float32),
            pltpu.VMEM_SHARED((NPAD, D), jnp.float32),
            pltpu.SemaphoreType.DMA,
            pltpu.SemaphoreType.DMA,
            pltpu.SemaphoreType.DMA,
            pltpu.SemaphoreType.DMA,
            pltpu.SemaphoreType.DMA,
            pltpu.SemaphoreType.DMA,
        ],
    )(y, src1d, dst1d, zeros2)


def _ep_body(acc_ref, xw_ref, dp_ref, b_ref, eye_ref, out_ref):
    deg = dp_ref[0] + dp_ref[1] + 1.0
    dis = lax.rsqrt(deg)
    invd = 1.0 / deg
    eye = eye_ref[...]
    acc = acc_ref[0] + acc_ref[1]
    for j in range(8):
        sl = slice(128 * j, 128 * (j + 1))
        dism = dis[j:j + 1, :] * eye
        invm = invd[j:j + 1, :] * eye
        h = _dot(dism, acc[sl, :]) + _dot(invm, xw_ref[sl, :]) + b_ref[0:1, :]
        out_ref[sl, :] = jnp.maximum(h, 0.0)


def _ep_call(accs, xw, dp3, b8, eye):
    return pl.pallas_call(
        _ep_body,
        grid=(10,),
        in_specs=[
            pl.BlockSpec((NC, 1024, D), lambda i: (0, i, 0)),
            pl.BlockSpec((1024, D), lambda i: (i, 0)),
            pl.BlockSpec((NC, 8, 128), lambda i: (0, i, 0)),
            pl.BlockSpec((8, D), lambda i: (0, 0)),
            pl.BlockSpec((D, D), lambda i: (0, 0)),
        ],
        out_specs=pl.BlockSpec((1024, D), lambda i: (i, 0)),
        out_shape=jax.ShapeDtypeStruct((NPAD, D), jnp.float32),
    )(accs, xw, dp3, b8, eye)


def kernel(x, edge_index, W, b):
    pad = N + jax.lax.rem(jnp.arange(EPAD - E, dtype=jnp.int32),
                          jnp.int32(NPAD - N))
    src1d = jnp.concatenate([edge_index[0], pad])
    dst1d = jnp.concatenate([edge_index[1], pad])
    x_pad = jnp.pad(x, ((0, NPAD - N), (0, 0)))
    ones1 = jnp.ones((CH,), jnp.float32)
    zeros1 = jnp.zeros((PER_TILE_N,), jnp.float32)
    zeros2 = jnp.zeros((ZR, D), jnp.float32)
    b8 = jnp.broadcast_to(b[None, :], (8, D))
    eye = jnp.eye(D, dtype=jnp.float32)

    dpart = _deg_call(dst1d, ones1, zeros1)
    dp3 = dpart.reshape(NC, NB, 128)
    xw = _mm_call(x_pad, W)
    y = _scale_call(xw, dp3, eye)
    accs = _edge_call(y, src1d, dst1d, zeros2)
    out = _ep_call(accs, xw, dp3, b8, eye)
    return out[:N]


# speedup vs baseline: 3.2748x; 1.0380x over previous
import functools

import jax
import jax.numpy as jnp
from jax import lax
from jax.experimental import pallas as pl
from jax.experimental.pallas import tpu as pltpu
from jax.experimental.pallas import tpu_sc as plsc

N = 10000
E = 320000
D = 128
NPAD = 10240
NB = NPAD // 128
CH = 128
NC = 2
NS = 16
NW = NC * NS
MAXCH = 80
EPAD = NW * MAXCH * CH
PER_TILE_N = NPAD // NS
ZR = 64

_mesh = functools.partial(
    plsc.VectorSubcoreMesh, core_axis_name="c", subcore_axis_name="s")


def _deg_body(dst_hbm, ones_hbm, zeros_hbm, out_hbm, idx0, idx1, idx2, idx3,
              onesv, stagev, deg_sp, semi0, semi1, semi2, semi3, sema0, sema1,
              sema2, sema3):
    c = lax.axis_index("c")
    s = lax.axis_index("s")
    wid = c * NS + s
    pltpu.sync_copy(ones_hbm, onesv)
    pltpu.sync_copy(zeros_hbm, stagev)
    pltpu.sync_copy(stagev, deg_sp.at[pl.ds(s * PER_TILE_N, PER_TILE_N)])
    plsc.subcore_barrier()
    base = wid * MAXCH * CH
    idxs = (idx0, idx1, idx2, idx3)
    semi = (semi0, semi1, semi2, semi3)
    sema = (sema0, sema1, sema2, sema3)

    def load(k, p):
        pltpu.async_copy(dst_hbm.at[pl.ds(base + k * CH, CH)], idxs[p],
                         semi[p])

    def add(k, p):
        pltpu.make_async_copy(dst_hbm.at[pl.ds(base + k * CH, CH)], idxs[p],
                              semi[p]).wait()
        pltpu.async_copy(onesv, deg_sp.at[idxs[p]], sema[p], add=True)

    def drain(p):
        pltpu.make_async_copy(onesv, deg_sp.at[idxs[p]], sema[p]).wait()

    for p in range(4):
        load(p, p)

    def body(gg, carry):
        k = 4 * gg
        for p in range(4):
            add(k + p, p)
        for p in range(4):
            drain(p)
            load(k + 4 + p, p)
        return carry

    lax.fori_loop(0, MAXCH // 4 - 1, body, 0)
    for p in range(4):
        add(MAXCH - 4 + p, p)
    for p in range(4):
        drain(p)
    plsc.subcore_barrier()
    pltpu.sync_copy(deg_sp.at[pl.ds(s * PER_TILE_N, PER_TILE_N)], stagev)
    pltpu.sync_copy(stagev, out_hbm.at[c, pl.ds(s * PER_TILE_N, PER_TILE_N)])


def _deg_call(dst1, ones1, zeros1):
    return pl.kernel(
        _deg_body,
        out_type=jax.ShapeDtypeStruct((NC, NPAD), jnp.float32),
        mesh=_mesh(),
        scratch_types=(
            [pltpu.VMEM((CH,), jnp.int32)] * 4
            + [pltpu.VMEM((CH,), jnp.float32),
               pltpu.VMEM((PER_TILE_N,), jnp.float32),
               pltpu.VMEM_SHARED((NPAD,), jnp.float32)]
            + [pltpu.SemaphoreType.DMA] * 8
        ),
    )(dst1, ones1, zeros1)


def _dot(a, b):
    return lax.dot_general(a, b, (((1,), (0,)), ((), ())),
                           preferred_element_type=jnp.float32)


def _mm_body(x_ref, w_ref, xw_ref):
    xw_ref[...] = lax.dot_general(x_ref[...], w_ref[...],
                                  (((1,), (1,)), ((), ())),
                                  preferred_element_type=jnp.float32)


def _mm_call(x, w):
    return pl.pallas_call(
        _mm_body,
        grid=(10,),
        in_specs=[
            pl.BlockSpec((1024, D), lambda i: (i, 0)),
            pl.BlockSpec((D, D), lambda i: (0, 0)),
        ],
        out_specs=pl.BlockSpec((1024, D), lambda i: (i, 0)),
        out_shape=jax.ShapeDtypeStruct((NPAD, D), jnp.float32),
    )(x, w)


def _scale_body(xw_ref, dp_ref, eye_ref, y_ref):
    deg = dp_ref[0] + dp_ref[1] + 1.0
    dis = lax.rsqrt(deg)
    eye = eye_ref[...]
    for j in range(8):
        diagm = dis[j:j + 1, :] * eye
        y_ref[128 * j:128 * (j + 1), :] = _dot(
            diagm, xw_ref[128 * j:128 * (j + 1), :])


def _scale_call(xw, dp3, eye):
    return pl.pallas_call(
        _scale_body,
        grid=(10,),
        in_specs=[
            pl.BlockSpec((1024, D), lambda i: (i, 0)),
            pl.BlockSpec((NC, 8, 128), lambda i: (0, i, 0)),
            pl.BlockSpec((D, D), lambda i: (0, 0)),
        ],
        out_specs=pl.BlockSpec((1024, D), lambda i: (i, 0)),
        out_shape=jax.ShapeDtypeStruct((NPAD, D), jnp.float32),
    )(xw, dp3, eye)


def _edge_body(y_hbm, src_hbm, dst_hbm, zeros_hbm, out_hbm, src0, src1, dst0,
               dst1, rows0, rows1, stage, acc_sp, semi0, semi1, semid0, semid1,
               semg0, semg1):
    c = lax.axis_index("c")
    s = lax.axis_index("s")
    wid = c * NS + s
    stages = (stage, rows0)
    pltpu.sync_copy(zeros_hbm, stage)
    for k in range(PER_TILE_N // ZR):
        pltpu.async_copy(stage, acc_sp.at[pl.ds(s * PER_TILE_N + k * ZR, ZR)],
                         semg0)
    for k in range(PER_TILE_N // ZR):
        pltpu.make_async_copy(
            stage, acc_sp.at[pl.ds(s * PER_TILE_N + k * ZR, ZR)], semg0).wait()
    plsc.subcore_barrier()

    base = wid * MAXCH * CH
    srcs = (src0, src1)
    dsts = (dst0, dst1)
    rows = (rows0, rows1)
    semis = (semi0, semi1)
    semid = (semid0, semid1)
    semg = (semg0, semg1)

    def load_src(k, p):
        pltpu.async_copy(src_hbm.at[pl.ds(base + k * CH, CH)], srcs[p],
                         semis[p])

    def load_dst(k, p):
        pltpu.async_copy(dst_hbm.at[pl.ds(base + k * CH, CH)], dsts[p],
                         semid[p])

    def gather(k, p):
        pltpu.make_async_copy(src_hbm.at[pl.ds(base + k * CH, CH)], srcs[p],
                              semis[p]).wait()
        pltpu.async_copy(y_hbm.at[srcs[p]], rows[p], semg[p])

    def wait_gather(p):
        pltpu.make_async_copy(y_hbm.at[srcs[p]], rows[p], semg[p]).wait()

    def scatter(k, p):
        pltpu.make_async_copy(dst_hbm.at[pl.ds(base + k * CH, CH)], dsts[p],
                              semid[p]).wait()
        pltpu.sync_copy(rows[p], acc_sp.at[dsts[p]], add=True)

    load_src(0, 0)
    load_dst(0, 0)
    load_src(1, 1)
    load_dst(1, 1)
    gather(0, 0)

    def body(gg, carry):
        k0 = 2 * gg
        k1 = 2 * gg + 1
        gather(k1, 1)
        wait_gather(0)
        load_src(k0 + 2, 0)
        scatter(k0, 0)
        load_dst(k0 + 2, 0)
        gather(k0 + 2, 0)
        wait_gather(1)
        load_src(k1 + 2, 1)
        scatter(k1, 1)
        load_dst(k1 + 2, 1)
        return carry

    lax.fori_loop(0, MAXCH // 2 - 1, body, 0)
    gather(MAXCH - 1, 1)
    wait_gather(0)
    scatter(MAXCH - 2, 0)
    wait_gather(1)
    scatter(MAXCH - 1, 1)
    plsc.subcore_barrier()
    stg = (stage, rows0.at[pl.ds(0, ZR)])
    semw = (semi0, semi1)
    nwr = PER_TILE_N // ZR

    def wr_slice(k):
        return pl.ds(s * PER_TILE_N + k * ZR, ZR)

    for k in range(nwr):
        p = k % 2
        if k >= 2:
            pltpu.make_async_copy(stg[p], out_hbm.at[c, wr_slice(k - 2)],
                                  semw[p]).wait()
        pltpu.sync_copy(acc_sp.at[wr_slice(k)], stg[p])
        pltpu.async_copy(stg[p], out_hbm.at[c, wr_slice(k)], semw[p])
    for k in range(nwr - 2, nwr):
        pltpu.make_async_copy(stg[k % 2], out_hbm.at[c, wr_slice(k)],
                              semw[k % 2]).wait()


def _edge_call(y, src1d, dst1d, zeros2):
    return pl.kernel(
        _edge_body,
        out_type=jax.ShapeDtypeStruct((NC, NPAD, D), jnp.float32),
        mesh=_mesh(),
        scratch_types=[
            pltpu.VMEM((CH,), jnp.int32),
            pltpu.VMEM((CH,), jnp.int32),
            pltpu.VMEM((CH,), jnp.int32),
            pltpu.VMEM((CH,), jnp.int32),
            pltpu.VMEM((CH, D), jnp.float32),
            pltpu.VMEM((CH, D), jnp.float32),
            pltpu.VMEM((ZR, D), jnp.float32),
            pltpu.VMEM_SHARED((NPAD, D), jnp.float32),
            pltpu.SemaphoreType.DMA,
            pltpu.SemaphoreType.DMA,
            pltpu.SemaphoreType.DMA,
            pltpu.SemaphoreType.DMA,
            pltpu.SemaphoreType.DMA,
            pltpu.SemaphoreType.DMA,
        ],
    )(y, src1d, dst1d, zeros2)


def _ep_body(acc_ref, xw_ref, dp_ref, b_ref, eye_ref, out_ref):
    deg = dp_ref[0] + dp_ref[1] + 1.0
    dis = lax.rsqrt(deg)
    invd = 1.0 / deg
    eye = eye_ref[...]
    acc = acc_ref[0] + acc_ref[1]
    for j in range(8):
        sl = slice(128 * j, 128 * (j + 1))
        dism = dis[j:j + 1, :] * eye
        invm = invd[j:j + 1, :] * eye
        h = _dot(dism, acc[sl, :]) + _dot(invm, xw_ref[sl, :]) + b_ref[0:1, :]
        out_ref[sl, :] = jnp.maximum(h, 0.0)


def _ep_call(accs, xw, dp3, b8, eye):
    return pl.pallas_call(
        _ep_body,
        grid=(10,),
        in_specs=[
            pl.BlockSpec((NC, 1024, D), lambda i: (0, i, 0)),
            pl.BlockSpec((1024, D), lambda i: (i, 0)),
            pl.BlockSpec((NC, 8, 128), lambda i: (0, i, 0)),
            pl.BlockSpec((8, D), lambda i: (0, 0)),
            pl.BlockSpec((D, D), lambda i: (0, 0)),
        ],
        out_specs=pl.BlockSpec((1024, D), lambda i: (i, 0)),
        out_shape=jax.ShapeDtypeStruct((N, D), jnp.float32),
    )(accs, xw, dp3, b8, eye)


def kernel(x, edge_index, W, b):
    pad = N + jax.lax.rem(jnp.arange(EPAD - E, dtype=jnp.int32),
                          jnp.int32(NPAD - N))
    src1d = jnp.concatenate([edge_index[0], pad])
    dst1d = jnp.concatenate([edge_index[1], pad])
    x_pad = jnp.pad(x, ((0, NPAD - N), (0, 0)))
    ones1 = jnp.ones((CH,), jnp.float32)
    zeros1 = jnp.zeros((PER_TILE_N,), jnp.float32)
    zeros2 = jnp.zeros((ZR, D), jnp.float32)
    b8 = jnp.broadcast_to(b[None, :], (8, D))
    eye = jnp.eye(D, dtype=jnp.float32)

    dpart = _deg_call(dst1d, ones1, zeros1)
    dp3 = dpart.reshape(NC, NB, 128)
    xw = _mm_call(x_pad, W)
    y = _scale_call(xw, dp3, eye)
    accs = _edge_call(y, src1d, dst1d, zeros2)
    return _ep_call(accs, xw, dp3, b8, eye)


# speedup vs baseline: 3.3871x; 1.0343x over previous
import functools

import jax
import jax.numpy as jnp
from jax import lax
from jax.experimental import pallas as pl
from jax.experimental.pallas import tpu as pltpu
from jax.experimental.pallas import tpu_sc as plsc

N = 10000
E = 320000
D = 128
NPAD = 10240
NB = NPAD // 128
CH = 128
NC = 2
NS = 16
NW = NC * NS
MAXCH = 80
EPAD = NW * MAXCH * CH
PER_TILE_N = NPAD // NS
ZR = 64

_mesh = functools.partial(
    plsc.VectorSubcoreMesh, core_axis_name="c", subcore_axis_name="s")


NDB = 8


def _deg_body(dst_hbm, ones_hbm, zeros_hbm, out_hbm, *refs):
    idxs = refs[0:NDB]
    onesv = refs[NDB]
    stagev = refs[NDB + 1]
    deg_sp = refs[NDB + 2]
    semi = refs[NDB + 3:NDB + 3 + NDB]
    sema = refs[NDB + 3 + NDB:]

    c = lax.axis_index("c")
    s = lax.axis_index("s")
    wid = c * NS + s
    pltpu.sync_copy(ones_hbm, onesv)
    pltpu.sync_copy(zeros_hbm, stagev)
    pltpu.sync_copy(stagev, deg_sp.at[pl.ds(s * PER_TILE_N, PER_TILE_N)])
    plsc.subcore_barrier()
    base = wid * MAXCH * CH

    def load(k, p):
        pltpu.async_copy(dst_hbm.at[pl.ds(base + k * CH, CH)], idxs[p],
                         semi[p])

    def add(k, p):
        pltpu.make_async_copy(dst_hbm.at[pl.ds(base + k * CH, CH)], idxs[p],
                              semi[p]).wait()
        pltpu.async_copy(onesv, deg_sp.at[idxs[p]], sema[p], add=True)

    def drain(p):
        pltpu.make_async_copy(onesv, deg_sp.at[idxs[p]], sema[p]).wait()

    for p in range(NDB):
        load(p, p)

    def body(gg, carry):
        k = NDB * gg
        for p in range(NDB):
            add(k + p, p)
        for p in range(NDB):
            drain(p)
            load(k + NDB + p, p)
        return carry

    lax.fori_loop(0, MAXCH // NDB - 1, body, 0)
    for p in range(NDB):
        add(MAXCH - NDB + p, p)
    for p in range(NDB):
        drain(p)
    plsc.subcore_barrier()
    pltpu.sync_copy(deg_sp.at[pl.ds(s * PER_TILE_N, PER_TILE_N)], stagev)
    pltpu.sync_copy(stagev, out_hbm.at[c, pl.ds(s * PER_TILE_N, PER_TILE_N)])


def _deg_call(dst1, ones1, zeros1):
    return pl.kernel(
        _deg_body,
        out_type=jax.ShapeDtypeStruct((NC, NPAD), jnp.float32),
        mesh=_mesh(),
        scratch_types=(
            [pltpu.VMEM((CH,), jnp.int32)] * NDB
            + [pltpu.VMEM((CH,), jnp.float32),
               pltpu.VMEM((PER_TILE_N,), jnp.float32),
               pltpu.VMEM_SHARED((NPAD,), jnp.float32)]
            + [pltpu.SemaphoreType.DMA] * (2 * NDB)
        ),
    )(dst1, ones1, zeros1)


def _dot(a, b):
    return lax.dot_general(a, b, (((1,), (0,)), ((), ())),
                           preferred_element_type=jnp.float32)


def _mm_body(x_ref, w_ref, xw_ref):
    xw_ref[...] = lax.dot_general(x_ref[...], w_ref[...],
                                  (((1,), (1,)), ((), ())),
                                  preferred_element_type=jnp.float32)


def _mm_call(x, w):
    return pl.pallas_call(
        _mm_body,
        grid=(10,),
        in_specs=[
            pl.BlockSpec((1024, D), lambda i: (i, 0)),
            pl.BlockSpec((D, D), lambda i: (0, 0)),
        ],
        out_specs=pl.BlockSpec((1024, D), lambda i: (i, 0)),
        out_shape=jax.ShapeDtypeStruct((NPAD, D), jnp.float32),
    )(x, w)


def _scale_body(xw_ref, dp_ref, eye_ref, y_ref):
    deg = dp_ref[0] + dp_ref[1] + 1.0
    dis = lax.rsqrt(deg)
    eye = eye_ref[...]
    for j in range(8):
        diagm = dis[j:j + 1, :] * eye
        y_ref[128 * j:128 * (j + 1), :] = _dot(
            diagm, xw_ref[128 * j:128 * (j + 1), :])


def _scale_call(xw, dp3, eye):
    return pl.pallas_call(
        _scale_body,
        grid=(10,),
        in_specs=[
            pl.BlockSpec((1024, D), lambda i: (i, 0)),
            pl.BlockSpec((NC, 8, 128), lambda i: (0, i, 0)),
            pl.BlockSpec((D, D), lambda i: (0, 0)),
        ],
        out_specs=pl.BlockSpec((1024, D), lambda i: (i, 0)),
        out_shape=jax.ShapeDtypeStruct((NPAD, D), jnp.float32),
    )(xw, dp3, eye)


def _edge_body(y_hbm, src_hbm, dst_hbm, zeros_hbm, out_hbm, src0, src1, dst0,
               dst1, rows0, rows1, stage, acc_sp, semi0, semi1, semid0, semid1,
               semg0, semg1):
    c = lax.axis_index("c")
    s = lax.axis_index("s")
    wid = c * NS + s
    stages = (stage, rows0)
    pltpu.sync_copy(zeros_hbm, stage)
    for k in range(PER_TILE_N // ZR):
        pltpu.async_copy(stage, acc_sp.at[pl.ds(s * PER_TILE_N + k * ZR, ZR)],
                         semg0)
    for k in range(PER_TILE_N // ZR):
        pltpu.make_async_copy(
            stage, acc_sp.at[pl.ds(s * PER_TILE_N + k * ZR, ZR)], semg0).wait()
    plsc.subcore_barrier()

    base = wid * MAXCH * CH
    srcs = (src0, src1)
    dsts = (dst0, dst1)
    rows = (rows0, rows1)
    semis = (semi0, semi1)
    semid = (semid0, semid1)
    semg = (semg0, semg1)

    def load_src(k, p):
        pltpu.async_copy(src_hbm.at[pl.ds(base + k * CH, CH)], srcs[p],
                         semis[p])

    def load_dst(k, p):
        pltpu.async_copy(dst_hbm.at[pl.ds(base + k * CH, CH)], dsts[p],
                         semid[p])

    def gather(k, p):
        pltpu.make_async_copy(src_hbm.at[pl.ds(base + k * CH, CH)], srcs[p],
                              semis[p]).wait()
        pltpu.async_copy(y_hbm.at[srcs[p]], rows[p], semg[p])

    def wait_gather(p):
        pltpu.make_async_copy(y_hbm.at[srcs[p]], rows[p], semg[p]).wait()

    def scatter(k, p):
        pltpu.make_async_copy(dst_hbm.at[pl.ds(base + k * CH, CH)], dsts[p],
                              semid[p]).wait()
        pltpu.sync_copy(rows[p], acc_sp.at[dsts[p]], add=True)

    load_src(0, 0)
    load_dst(0, 0)
    load_src(1, 1)
    load_dst(1, 1)
    gather(0, 0)

    def body(gg, carry):
        k0 = 2 * gg
        k1 = 2 * gg + 1
        gather(k1, 1)
        wait_gather(0)
        load_src(k0 + 2, 0)
        scatter(k0, 0)
        load_dst(k0 + 2, 0)
        gather(k0 + 2, 0)
        wait_gather(1)
        load_src(k1 + 2, 1)
        scatter(k1, 1)
        load_dst(k1 + 2, 1)
        return carry

    lax.fori_loop(0, MAXCH // 2 - 1, body, 0)
    gather(MAXCH - 1, 1)
    wait_gather(0)
    scatter(MAXCH - 2, 0)
    wait_gather(1)
    scatter(MAXCH - 1, 1)
    plsc.subcore_barrier()
    stg = (stage, rows0.at[pl.ds(0, ZR)])
    semw = (semi0, semi1)
    nwr = PER_TILE_N // ZR

    def wr_slice(k):
        return pl.ds(s * PER_TILE_N + k * ZR, ZR)

    for k in range(nwr):
        p = k % 2
        if k >= 2:
            pltpu.make_async_copy(stg[p], out_hbm.at[c, wr_slice(k - 2)],
                                  semw[p]).wait()
        pltpu.sync_copy(acc_sp.at[wr_slice(k)], stg[p])
        pltpu.async_copy(stg[p], out_hbm.at[c, wr_slice(k)], semw[p])
    for k in range(nwr - 2, nwr):
        pltpu.make_async_copy(stg[k % 2], out_hbm.at[c, wr_slice(k)],
                              semw[k % 2]).wait()


def _edge_call(y, src1d, dst1d, zeros2):
    return pl.kernel(
        _edge_body,
        out_type=jax.ShapeDtypeStruct((NC, NPAD, D), jnp.float32),
        mesh=_mesh(),
        scratch_types=[
            pltpu.VMEM((CH,), jnp.int32),
            pltpu.VMEM((CH,), jnp.int32),
            pltpu.VMEM((CH,), jnp.int32),
            pltpu.VMEM((CH,), jnp.int32),
            pltpu.VMEM((CH, D), jnp.float32),
            pltpu.VMEM((CH, D), jnp.float32),
            pltpu.VMEM((ZR, D), jnp.float32),
            pltpu.VMEM_SHARED((NPAD, D), jnp.float32),
            pltpu.SemaphoreType.DMA,
            pltpu.SemaphoreType.DMA,
            pltpu.SemaphoreType.DMA,
            pltpu.SemaphoreType.DMA,
            pltpu.SemaphoreType.DMA,
            pltpu.SemaphoreType.DMA,
        ],
    )(y, src1d, dst1d, zeros2)


def _ep_body(acc_ref, xw_ref, dp_ref, b_ref, eye_ref, out_ref):
    deg = dp_ref[0] + dp_ref[1] + 1.0
    dis = lax.rsqrt(deg)
    invd = 1.0 / deg
    eye = eye_ref[...]
    acc = acc_ref[0] + acc_ref[1]
    for j in range(8):
        sl = slice(128 * j, 128 * (j + 1))
        dism = dis[j:j + 1, :] * eye
        invm = invd[j:j + 1, :] * eye
        h = _dot(dism, acc[sl, :]) + _dot(invm, xw_ref[sl, :]) + b_ref[0:1, :]
        out_ref[sl, :] = jnp.maximum(h, 0.0)


def _ep_call(accs, xw, dp3, b8, eye):
    return pl.pallas_call(
        _ep_body,
        grid=(10,),
        in_specs=[
            pl.BlockSpec((NC, 1024, D), lambda i: (0, i, 0)),
            pl.BlockSpec((1024, D), lambda i: (i, 0)),
            pl.BlockSpec((NC, 8, 128), lambda i: (0, i, 0)),
            pl.BlockSpec((8, D), lambda i: (0, 0)),
            pl.BlockSpec((D, D), lambda i: (0, 0)),
        ],
        out_specs=pl.BlockSpec((1024, D), lambda i: (i, 0)),
        out_shape=jax.ShapeDtypeStruct((N, D), jnp.float32),
    )(accs, xw, dp3, b8, eye)


def kernel(x, edge_index, W, b):
    pad = N + jax.lax.rem(jnp.arange(EPAD - E, dtype=jnp.int32),
                          jnp.int32(NPAD - N))
    src1d = jnp.concatenate([edge_index[0], pad])
    dst1d = jnp.concatenate([edge_index[1], pad])
    x_pad = jnp.pad(x, ((0, NPAD - N), (0, 0)))
    ones1 = jnp.ones((CH,), jnp.float32)
    zeros1 = jnp.zeros((PER_TILE_N,), jnp.float32)
    zeros2 = jnp.zeros((ZR, D), jnp.float32)
    b8 = jnp.broadcast_to(b[None, :], (8, D))
    eye = jnp.eye(D, dtype=jnp.float32)

    dpart = _deg_call(dst1d, ones1, zeros1)
    dp3 = dpart.reshape(NC, NB, 128)
    xw = _mm_call(x_pad, W)
    y = _scale_call(xw, dp3, eye)
    accs = _edge_call(y, src1d, dst1d, zeros2)
    return _ep_call(accs, xw, dp3, b8, eye)


# speedup vs baseline: 3.4105x; 1.0069x over previous
import functools

import jax
import jax.numpy as jnp
from jax import lax
from jax.experimental import pallas as pl
from jax.experimental.pallas import tpu as pltpu
from jax.experimental.pallas import tpu_sc as plsc

N = 10000
E = 320000
D = 128
NPAD = 10240
NB = NPAD // 128
CH = 128
NC = 2
NS = 16
NW = NC * NS
MAXCH = 80
EPAD = NW * MAXCH * CH
PER_TILE_N = NPAD // NS
ZR = 64

_mesh = functools.partial(
    plsc.VectorSubcoreMesh, core_axis_name="c", subcore_axis_name="s")


NDB = 8


def _deg_body(dst_hbm, ones_hbm, zeros_hbm, out_hbm, *refs):
    idxs = refs[0:NDB]
    onesv = refs[NDB]
    stagev = refs[NDB + 1]
    deg_sp = refs[NDB + 2]
    semi = refs[NDB + 3:NDB + 3 + NDB]
    sema = refs[NDB + 3 + NDB:]

    c = lax.axis_index("c")
    s = lax.axis_index("s")
    wid = c * NS + s
    pltpu.sync_copy(ones_hbm, onesv)
    pltpu.sync_copy(zeros_hbm, stagev)
    pltpu.sync_copy(stagev, deg_sp.at[pl.ds(s * PER_TILE_N, PER_TILE_N)])
    plsc.subcore_barrier()
    base = wid * MAXCH * 2 * CH

    def load(k, p):
        pltpu.async_copy(dst_hbm.at[pl.ds(base + (2 * k + 1) * CH, CH)],
                         idxs[p], semi[p])

    def add(k, p):
        pltpu.make_async_copy(dst_hbm.at[pl.ds(base + (2 * k + 1) * CH, CH)],
                              idxs[p], semi[p]).wait()
        pltpu.async_copy(onesv, deg_sp.at[idxs[p]], sema[p], add=True)

    def drain(p):
        pltpu.make_async_copy(onesv, deg_sp.at[idxs[p]], sema[p]).wait()

    for p in range(NDB):
        load(p, p)

    def body(gg, carry):
        k = NDB * gg
        for p in range(NDB):
            add(k + p, p)
        for p in range(NDB):
            drain(p)
            load(k + NDB + p, p)
        return carry

    lax.fori_loop(0, MAXCH // NDB - 1, body, 0)
    for p in range(NDB):
        add(MAXCH - NDB + p, p)
    for p in range(NDB):
        drain(p)
    plsc.subcore_barrier()
    pltpu.sync_copy(deg_sp.at[pl.ds(s * PER_TILE_N, PER_TILE_N)], stagev)
    pltpu.sync_copy(stagev, out_hbm.at[c, pl.ds(s * PER_TILE_N, PER_TILE_N)])


def _deg_call(dst1, ones1, zeros1):
    return pl.kernel(
        _deg_body,
        out_type=jax.ShapeDtypeStruct((NC, NPAD), jnp.float32),
        mesh=_mesh(),
        scratch_types=(
            [pltpu.VMEM((CH,), jnp.int32)] * NDB
            + [pltpu.VMEM((CH,), jnp.float32),
               pltpu.VMEM((PER_TILE_N,), jnp.float32),
               pltpu.VMEM_SHARED((NPAD,), jnp.float32)]
            + [pltpu.SemaphoreType.DMA] * (2 * NDB)
        ),
    )(dst1, ones1, zeros1)


def _dot(a, b):
    return lax.dot_general(a, b, (((1,), (0,)), ((), ())),
                           preferred_element_type=jnp.float32)


def _mm_body(x_ref, w_ref, xw_ref):
    xw_ref[...] = lax.dot_general(x_ref[...], w_ref[...],
                                  (((1,), (1,)), ((), ())),
                                  preferred_element_type=jnp.float32)


def _mm_call(x, w):
    return pl.pallas_call(
        _mm_body,
        grid=(10,),
        in_specs=[
            pl.BlockSpec((1024, D), lambda i: (i, 0)),
            pl.BlockSpec((D, D), lambda i: (0, 0)),
        ],
        out_specs=pl.BlockSpec((1024, D), lambda i: (i, 0)),
        out_shape=jax.ShapeDtypeStruct((NPAD, D), jnp.float32),
    )(x, w)


def _scale_body(xw_ref, dp_ref, eye_ref, y_ref):
    deg = dp_ref[0] + dp_ref[1] + 1.0
    dis = lax.rsqrt(deg)
    eye = eye_ref[...]
    for j in range(8):
        diagm = dis[j:j + 1, :] * eye
        y_ref[128 * j:128 * (j + 1), :] = _dot(
            diagm, xw_ref[128 * j:128 * (j + 1), :])


def _scale_call(xw, dp3, eye):
    return pl.pallas_call(
        _scale_body,
        grid=(10,),
        in_specs=[
            pl.BlockSpec((1024, D), lambda i: (i, 0)),
            pl.BlockSpec((NC, 8, 128), lambda i: (0, i, 0)),
            pl.BlockSpec((D, D), lambda i: (0, 0)),
        ],
        out_specs=pl.BlockSpec((1024, D), lambda i: (i, 0)),
        out_shape=jax.ShapeDtypeStruct((NPAD, D), jnp.float32),
    )(xw, dp3, eye)


def _edge_body(y_hbm, src_hbm, dst_hbm, zeros_hbm, out_hbm, src0, src1, dst0,
               dst1, rows0, rows1, stage, acc_sp, semi0, semi1, semid0, semid1,
               semg0, semg1):
    c = lax.axis_index("c")
    s = lax.axis_index("s")
    wid = c * NS + s
    stages = (stage, rows0)
    pltpu.sync_copy(zeros_hbm, stage)
    for k in range(PER_TILE_N // ZR):
        pltpu.async_copy(stage, acc_sp.at[pl.ds(s * PER_TILE_N + k * ZR, ZR)],
                         semg0)
    for k in range(PER_TILE_N // ZR):
        pltpu.make_async_copy(
            stage, acc_sp.at[pl.ds(s * PER_TILE_N + k * ZR, ZR)], semg0).wait()
    plsc.subcore_barrier()

    base = wid * MAXCH * 2 * CH
    srcs = (src0, src1)
    dsts = (dst0, dst1)
    rows = (rows0, rows1)
    semis = (semi0, semi1)
    semid = (semid0, semid1)
    semg = (semg0, semg1)

    def load_src(k, p):
        pltpu.async_copy(src_hbm.at[pl.ds(base + 2 * k * CH, CH)], srcs[p],
                         semis[p])

    def load_dst(k, p):
        pltpu.async_copy(dst_hbm.at[pl.ds(base + (2 * k + 1) * CH, CH)],
                         dsts[p], semid[p])

    def gather(k, p):
        pltpu.make_async_copy(src_hbm.at[pl.ds(base + 2 * k * CH, CH)],
                              srcs[p], semis[p]).wait()
        pltpu.async_copy(y_hbm.at[srcs[p]], rows[p], semg[p])

    def wait_gather(p):
        pltpu.make_async_copy(y_hbm.at[srcs[p]], rows[p], semg[p]).wait()

    def scatter(k, p):
        pltpu.make_async_copy(dst_hbm.at[pl.ds(base + (2 * k + 1) * CH, CH)],
                              dsts[p], semid[p]).wait()
        pltpu.sync_copy(rows[p], acc_sp.at[dsts[p]], add=True)

    load_src(0, 0)
    load_dst(0, 0)
    load_src(1, 1)
    load_dst(1, 1)
    gather(0, 0)

    def body(gg, carry):
        k0 = 2 * gg
        k1 = 2 * gg + 1
        gather(k1, 1)
        wait_gather(0)
        load_src(k0 + 2, 0)
        scatter(k0, 0)
        load_dst(k0 + 2, 0)
        gather(k0 + 2, 0)
        wait_gather(1)
        load_src(k1 + 2, 1)
        scatter(k1, 1)
        load_dst(k1 + 2, 1)
        return carry

    lax.fori_loop(0, MAXCH // 2 - 1, body, 0)
    gather(MAXCH - 1, 1)
    wait_gather(0)
    scatter(MAXCH - 2, 0)
    wait_gather(1)
    scatter(MAXCH - 1, 1)
    plsc.subcore_barrier()
    stg = (stage, rows0.at[pl.ds(0, ZR)])
    semw = (semi0, semi1)
    nwr = PER_TILE_N // ZR

    def wr_slice(k):
        return pl.ds(s * PER_TILE_N + k * ZR, ZR)

    for k in range(nwr):
        p = k % 2
        if k >= 2:
            pltpu.make_async_copy(stg[p], out_hbm.at[c, wr_slice(k - 2)],
                                  semw[p]).wait()
        pltpu.sync_copy(acc_sp.at[wr_slice(k)], stg[p])
        pltpu.async_copy(stg[p], out_hbm.at[c, wr_slice(k)], semw[p])
    for k in range(nwr - 2, nwr):
        pltpu.make_async_copy(stg[k % 2], out_hbm.at[c, wr_slice(k)],
                              semw[k % 2]).wait()


def _edge_call(y, src1d, dst1d, zeros2):
    return pl.kernel(
        _edge_body,
        out_type=jax.ShapeDtypeStruct((NC, NPAD, D), jnp.float32),
        mesh=_mesh(),
        scratch_types=[
            pltpu.VMEM((CH,), jnp.int32),
            pltpu.VMEM((CH,), jnp.int32),
            pltpu.VMEM((CH,), jnp.int32),
            pltpu.VMEM((CH,), jnp.int32),
            pltpu.VMEM((CH, D), jnp.float32),
            pltpu.VMEM((CH, D), jnp.float32),
            pltpu.VMEM((ZR, D), jnp.float32),
            pltpu.VMEM_SHARED((NPAD, D), jnp.float32),
            pltpu.SemaphoreType.DMA,
            pltpu.SemaphoreType.DMA,
            pltpu.SemaphoreType.DMA,
            pltpu.SemaphoreType.DMA,
            pltpu.SemaphoreType.DMA,
            pltpu.SemaphoreType.DMA,
        ],
    )(y, src1d, dst1d, zeros2)


def _ep_body(acc_ref, xw_ref, dp_ref, b_ref, eye_ref, out_ref):
    deg = dp_ref[0] + dp_ref[1] + 1.0
    dis = lax.rsqrt(deg)
    invd = 1.0 / deg
    eye = eye_ref[...]
    acc = acc_ref[0] + acc_ref[1]
    for j in range(8):
        sl = slice(128 * j, 128 * (j + 1))
        dism = dis[j:j + 1, :] * eye
        invm = invd[j:j + 1, :] * eye
        h = _dot(dism, acc[sl, :]) + _dot(invm, xw_ref[sl, :]) + b_ref[0:1, :]
        out_ref[sl, :] = jnp.maximum(h, 0.0)


def _ep_call(accs, xw, dp3, b8, eye):
    return pl.pallas_call(
        _ep_body,
        grid=(10,),
        in_specs=[
            pl.BlockSpec((NC, 1024, D), lambda i: (0, i, 0)),
            pl.BlockSpec((1024, D), lambda i: (i, 0)),
            pl.BlockSpec((NC, 8, 128), lambda i: (0, i, 0)),
            pl.BlockSpec((8, D), lambda i: (0, 0)),
            pl.BlockSpec((D, D), lambda i: (0, 0)),
        ],
        out_specs=pl.BlockSpec((1024, D), lambda i: (i, 0)),
        out_shape=jax.ShapeDtypeStruct((N, D), jnp.float32),
    )(accs, xw, dp3, b8, eye)


def kernel(x, edge_index, W, b):
    ei3 = jnp.swapaxes(edge_index.reshape(2, E // CH, CH), 0, 1)
    pad = N + jax.lax.rem(jnp.arange((EPAD - E) * 2, dtype=jnp.int32),
                          jnp.int32(NPAD - N))
    eiflat = jnp.concatenate([ei3.reshape(-1), pad])
    x_pad = jnp.pad(x, ((0, NPAD - N), (0, 0)))
    ones1 = jnp.ones((CH,), jnp.float32)
    zeros1 = jnp.zeros((PER_TILE_N,), jnp.float32)
    zeros2 = jnp.zeros((ZR, D), jnp.float32)
    b8 = jnp.broadcast_to(b[None, :], (8, D))
    eye = jnp.eye(D, dtype=jnp.float32)

    dpart = _deg_call(eiflat, ones1, zeros1)
    dp3 = dpart.reshape(NC, NB, 128)
    xw = _mm_call(x_pad, W)
    y = _scale_call(xw, dp3, eye)
    accs = _edge_call(y, eiflat, eiflat, zeros2)
    return _ep_call(accs, xw, dp3, b8, eye)
